# baseline (speedup 1.0000x reference)
import functools

import jax
import jax.numpy as jnp
from jax import lax
from jax.experimental import pallas as pl
from jax.experimental.pallas import tpu as pltpu
from jax.experimental.pallas import tpu_sc as plsc

N_NODES = 10000
N_EDGES = 320000
D = 128

NC = 2
NS = 16
NW = NC * NS

CHUNK = 128
NROWS = N_EDGES // CHUNK
CHUNKS_T = NROWS // NW
XBASE = NW * CHUNKS_T
NEXTRA = NROWS - XBASE
IBLK = 16
BLOCKS = [IBLK] * (CHUNKS_T // IBLK) + (
    [CHUNKS_T % IBLK] if CHUNKS_T % IBLK else [])
N_PAD = 10112
ZROWS = N_PAD // NS


def _sc_aggregate(edges3, x):
    mesh = plsc.VectorSubcoreMesh(core_axis_name="c", subcore_axis_name="s")

    @functools.partial(
        pl.kernel,
        out_type=jax.ShapeDtypeStruct((NC * N_PAD, D), jnp.float32),
        mesh=mesh,
        compiler_params=pltpu.CompilerParams(use_tc_tiling_on_sc=False),
        scratch_types=[
            pltpu.VMEM((2, IBLK, CHUNK), jnp.int32),
            pltpu.VMEM((2, IBLK, CHUNK), jnp.int32),
            pltpu.VMEM((CHUNK, D), jnp.float32),
            pltpu.VMEM((CHUNK, D), jnp.float32),
            pltpu.VMEM_SHARED((N_PAD, D), jnp.float32),
            pltpu.SemaphoreType.DMA,
            pltpu.SemaphoreType.DMA,
            pltpu.SemaphoreType.DMA,
        ],
    )
    def sc_kernel(e_hbm, x_hbm, out_hbm,
                  src_v, dst_v, bufa, bufb, agg, sema, semb, semi):
        cid = lax.axis_index("c")
        sid = lax.axis_index("s")
        tid = cid * NS + sid
        src_hbm = e_hbm.at[0]
        dst_hbm = e_hbm.at[1]

        @pl.loop(0, CHUNK)
        def _(i):
            for k in range(D // 16):
                bufa[i, pl.ds(k * 16, 16)] = jnp.zeros((16,), jnp.float32)

        zbase = sid * ZROWS
        nfull = ZROWS // CHUNK
        for z in range(nfull):
            pltpu.sync_copy(bufa, agg.at[pl.ds(zbase + z * CHUNK, CHUNK)])
        rem = ZROWS - nfull * CHUNK
        if rem:
            pltpu.sync_copy(bufa.at[pl.ds(0, rem)],
                            agg.at[pl.ds(zbase + nfull * CHUNK, rem)])
        plsc.subcore_barrier()

        base = tid * CHUNKS_T
        pltpu.sync_copy(src_hbm.at[pl.ds(base, IBLK)], src_v.at[0])
        pltpu.sync_copy(dst_hbm.at[pl.ds(base, IBLK)], dst_v.at[0])

        for blk, blen in enumerate(BLOCKS):
            cur = blk % 2
            nxt = 1 - cur
            if blk + 1 < len(BLOCKS):
                hs = pltpu.async_copy(
                    src_hbm.at[pl.ds(base + (blk + 1) * IBLK, IBLK)],
                    src_v.at[nxt], semi)
                hd = pltpu.async_copy(
                    dst_hbm.at[pl.ds(base + (blk + 1) * IBLK, IBLK)],
                    dst_v.at[nxt], semi)
            sv = src_v.at[cur]
            dv = dst_v.at[cur]
            pltpu.async_copy(x_hbm.at[sv.at[0]], bufa, sema)

            @pl.loop(0, blen // 2)
            def _(g):
                j0 = g * 2
                j1 = j0 + 1
                pltpu.async_copy(x_hbm.at[sv.at[j1]], bufb, semb)
                pltpu.make_async_copy(x_hbm.at[sv.at[j0]], bufa, sema).wait()
                pltpu.sync_copy(bufa, agg.at[dv.at[j0]], add=True)

                @pl.when(j1 + 1 < blen)
                def _():
                    pltpu.async_copy(x_hbm.at[sv.at[j1 + 1]], bufa, sema)

                pltpu.make_async_copy(x_hbm.at[sv.at[j1]], bufb, semb).wait()
                pltpu.sync_copy(bufb, agg.at[dv.at[j1]], add=True)

            if blk + 1 < len(BLOCKS):
                hs.wait()
                hd.wait()

        @pl.when(tid < NEXTRA)
        def _():
            pltpu.sync_copy(src_hbm.at[pl.ds(XBASE, NEXTRA)],
                            src_v.at[0, pl.ds(0, NEXTRA)])
            pltpu.sync_copy(dst_hbm.at[pl.ds(XBASE, NEXTRA)],
                            dst_v.at[0, pl.ds(0, NEXTRA)])
            pltpu.sync_copy(x_hbm.at[src_v.at[0, tid]], bufa)
            pltpu.sync_copy(bufa, agg.at[dst_v.at[0, tid]], add=True)

        plsc.subcore_barrier()

        obase = sid * ZROWS
        pltpu.sync_copy(agg.at[pl.ds(obase, ZROWS)],
                        out_hbm.at[pl.ds(cid * N_PAD + obase, ZROWS)])

    return sc_kernel(edges3, x)


def _tc_xw_body(x_ref, w_ref, b_ref, o_ref):
    o_ref[...] = lax.dot_general(
        x_ref[...], w_ref[...],
        dimension_numbers=(((1,), (1,)), ((), ())),
        preferred_element_type=jnp.float32,
    ) + b_ref[...]


def _tc_out_body(xw_ref, a0_ref, a1_ref, w_ref, o_ref):
    h = a0_ref[0] + a1_ref[0]
    o_ref[...] = lax.dot_general(
        h, w_ref[...],
        dimension_numbers=(((1,), (1,)), ((), ())),
        preferred_element_type=jnp.float32,
    ) + xw_ref[...]


def kernel(x, edge_index, W, b):
    edges3 = edge_index.reshape(2, NROWS, CHUNK)

    BM = 1000
    nb = N_NODES // BM
    xw = pl.pallas_call(
        _tc_xw_body,
        grid=(nb,),
        in_specs=[
            pl.BlockSpec((BM, D), lambda i: (i, 0)),
            pl.BlockSpec((D, D), lambda i: (0, 0)),
            pl.BlockSpec((1, D), lambda i: (0, 0)),
        ],
        out_specs=pl.BlockSpec((BM, D), lambda i: (i, 0)),
        out_shape=jax.ShapeDtypeStruct((N_NODES, D), jnp.float32),
    )(x, W, b.reshape(1, D))

    agg = _sc_aggregate(edges3, x).reshape(NC, N_PAD, D)

    out = pl.pallas_call(
        _tc_out_body,
        grid=(nb,),
        in_specs=[
            pl.BlockSpec((BM, D), lambda i: (i, 0)),
            pl.BlockSpec((1, BM, D), lambda i: (0, i, 0)),
            pl.BlockSpec((1, BM, D), lambda i: (1, i, 0)),
            pl.BlockSpec((D, D), lambda i: (0, 0)),
        ],
        out_specs=pl.BlockSpec((BM, D), lambda i: (i, 0)),
        out_shape=jax.ShapeDtypeStruct((N_NODES, D), jnp.float32),
    )(xw, agg, agg, W)
    return out


# speedup vs baseline: 1.0243x; 1.0243x over previous
import functools

import jax
import jax.numpy as jnp
from jax import lax
from jax.experimental import pallas as pl
from jax.experimental.pallas import tpu as pltpu
from jax.experimental.pallas import tpu_sc as plsc

N_NODES = 10000
N_EDGES = 320000
D = 128

NC = 2
NS = 16
NW = NC * NS

CHUNK = 128
NROWS = N_EDGES // CHUNK
CHUNKS_T = NROWS // NW
XBASE = NW * CHUNKS_T
NEXTRA = NROWS - XBASE
IBLK = 16
BLOCKS = [IBLK] * (CHUNKS_T // IBLK) + (
    [CHUNKS_T % IBLK] if CHUNKS_T % IBLK else [])
N_PAD = 10112
ZROWS = N_PAD // NS


def _sc_aggregate(edges3, x):
    mesh = plsc.VectorSubcoreMesh(core_axis_name="c", subcore_axis_name="s")

    @functools.partial(
        pl.kernel,
        out_type=jax.ShapeDtypeStruct((NC * N_PAD, D), jnp.float32),
        mesh=mesh,
        compiler_params=pltpu.CompilerParams(use_tc_tiling_on_sc=False),
        scratch_types=[
            pltpu.VMEM((2, IBLK, CHUNK), jnp.int32),
            pltpu.VMEM((2, IBLK, CHUNK), jnp.int32),
            pltpu.VMEM((CHUNK, D), jnp.float32),
            pltpu.VMEM((CHUNK, D), jnp.float32),
            pltpu.VMEM_SHARED((N_PAD, D), jnp.float32),
            pltpu.SemaphoreType.DMA,
            pltpu.SemaphoreType.DMA,
            pltpu.SemaphoreType.DMA,
        ],
    )
    def sc_kernel(e_hbm, x_hbm, out_hbm,
                  src_v, dst_v, bufa, bufb, agg, sema, semb, semi):
        cid = lax.axis_index("c")
        sid = lax.axis_index("s")
        tid = cid * NS + sid
        src_hbm = e_hbm.at[0]
        dst_hbm = e_hbm.at[1]

        base = tid * CHUNKS_T
        pltpu.sync_copy(src_hbm.at[pl.ds(base, IBLK)], src_v.at[0])
        pltpu.sync_copy(dst_hbm.at[pl.ds(base, IBLK)], dst_v.at[0])
        pltpu.async_copy(x_hbm.at[src_v.at[0, 0]], bufa, sema)

        @pl.loop(0, CHUNK)
        def _(i):
            for k in range(D // 16):
                bufb[i, pl.ds(k * 16, 16)] = jnp.zeros((16,), jnp.float32)

        zbase = sid * ZROWS
        nfull = ZROWS // CHUNK
        for z in range(nfull):
            pltpu.sync_copy(bufb, agg.at[pl.ds(zbase + z * CHUNK, CHUNK)])
        rem = ZROWS - nfull * CHUNK
        if rem:
            pltpu.sync_copy(bufb.at[pl.ds(0, rem)],
                            agg.at[pl.ds(zbase + nfull * CHUNK, rem)])
        plsc.subcore_barrier()

        for blk, blen in enumerate(BLOCKS):
            cur = blk % 2
            nxt = 1 - cur
            if blk + 1 < len(BLOCKS):
                hs = pltpu.async_copy(
                    src_hbm.at[pl.ds(base + (blk + 1) * IBLK, IBLK)],
                    src_v.at[nxt], semi)
                hd = pltpu.async_copy(
                    dst_hbm.at[pl.ds(base + (blk + 1) * IBLK, IBLK)],
                    dst_v.at[nxt], semi)
            sv = src_v.at[cur]
            dv = dst_v.at[cur]
            if blk > 0:
                pltpu.async_copy(x_hbm.at[sv.at[0]], bufa, sema)

            @pl.loop(0, blen // 2)
            def _(g):
                j0 = g * 2
                j1 = j0 + 1
                pltpu.async_copy(x_hbm.at[sv.at[j1]], bufb, semb)
                pltpu.make_async_copy(x_hbm.at[sv.at[j0]], bufa, sema).wait()
                pltpu.sync_copy(bufa, agg.at[dv.at[j0]], add=True)

                @pl.when(j1 + 1 < blen)
                def _():
                    pltpu.async_copy(x_hbm.at[sv.at[j1 + 1]], bufa, sema)

                pltpu.make_async_copy(x_hbm.at[sv.at[j1]], bufb, semb).wait()
                pltpu.sync_copy(bufb, agg.at[dv.at[j1]], add=True)

            if blk + 1 < len(BLOCKS):
                hs.wait()
                hd.wait()

        @pl.when(tid < NEXTRA)
        def _():
            pltpu.sync_copy(src_hbm.at[pl.ds(XBASE, NEXTRA)],
                            src_v.at[0, pl.ds(0, NEXTRA)])
            pltpu.sync_copy(dst_hbm.at[pl.ds(XBASE, NEXTRA)],
                            dst_v.at[0, pl.ds(0, NEXTRA)])
            pltpu.sync_copy(x_hbm.at[src_v.at[0, tid]], bufa)
            pltpu.sync_copy(bufa, agg.at[dst_v.at[0, tid]], add=True)

        plsc.subcore_barrier()

        obase = sid * ZROWS
        hbase = cid * N_PAD + obase
        pieces = [(p * CHUNK, CHUNK) for p in range(ZROWS // CHUNK)]
        if ZROWS % CHUNK:
            pieces.append((ZROWS - ZROWS % CHUNK, ZROWS % CHUNK))
        bufp = (bufa, bufb)
        semp = (sema, semb)
        for p, (off, ln) in enumerate(pieces):
            pb = bufp[p % 2].at[pl.ds(0, ln)]
            if p >= 2:
                poff, pln = pieces[p - 2]
                pltpu.make_async_copy(
                    bufp[p % 2].at[pl.ds(0, pln)],
                    out_hbm.at[pl.ds(hbase + poff, pln)], semi).wait()
            pltpu.sync_copy(agg.at[pl.ds(obase + off, ln)], pb)
            pltpu.async_copy(pb, out_hbm.at[pl.ds(hbase + off, ln)], semi)
        for p in range(max(len(pieces) - 2, 0), len(pieces)):
            off, ln = pieces[p]
            pltpu.make_async_copy(
                bufp[p % 2].at[pl.ds(0, ln)],
                out_hbm.at[pl.ds(hbase + off, ln)], semi).wait()

    return sc_kernel(edges3, x)


def _tc_body(x_ref, a0_ref, a1_ref, w_ref, b_ref, o_ref):
    h = x_ref[...] + a0_ref[0] + a1_ref[0]
    o_ref[...] = lax.dot_general(
        h, w_ref[...],
        dimension_numbers=(((1,), (1,)), ((), ())),
        preferred_element_type=jnp.float32,
    ) + b_ref[...]


def kernel(x, edge_index, W, b):
    edges3 = edge_index.reshape(2, NROWS, CHUNK)

    agg = _sc_aggregate(edges3, x).reshape(NC, N_PAD, D)

    BM = 1000
    nb = N_NODES // BM
    out = pl.pallas_call(
        _tc_body,
        grid=(nb,),
        in_specs=[
            pl.BlockSpec((BM, D), lambda i: (i, 0)),
            pl.BlockSpec((1, BM, D), lambda i: (0, i, 0)),
            pl.BlockSpec((1, BM, D), lambda i: (1, i, 0)),
            pl.BlockSpec((D, D), lambda i: (0, 0)),
            pl.BlockSpec((1, D), lambda i: (0, 0)),
        ],
        out_specs=pl.BlockSpec((BM, D), lambda i: (i, 0)),
        out_shape=jax.ShapeDtypeStruct((N_NODES, D), jnp.float32),
    )(x, agg, agg, W, b.reshape(1, D))
    return out


# speedup vs baseline: 1.0463x; 1.0214x over previous
import functools

import jax
import jax.numpy as jnp
from jax import lax
from jax.experimental import pallas as pl
from jax.experimental.pallas import tpu as pltpu
from jax.experimental.pallas import tpu_sc as plsc

N_NODES = 10000
N_EDGES = 320000
D = 128

NC = 2
NS = 16
NW = NC * NS

CHUNK = 128
NROWS = N_EDGES // CHUNK
CHUNKS_T = NROWS // NW
XBASE = NW * CHUNKS_T
NEXTRA = NROWS - XBASE
IBLK = 16
BLOCKS = [IBLK] * (CHUNKS_T // IBLK) + (
    [CHUNKS_T % IBLK] if CHUNKS_T % IBLK else [])
N_PAD = 10112
ZROWS = N_PAD // NS


def _sc_aggregate(edges3, x):
    mesh = plsc.VectorSubcoreMesh(core_axis_name="c", subcore_axis_name="s")

    @functools.partial(
        pl.kernel,
        out_type=jax.ShapeDtypeStruct((NC * N_PAD, D), jnp.float32),
        mesh=mesh,
        compiler_params=pltpu.CompilerParams(use_tc_tiling_on_sc=False),
        scratch_types=[
            pltpu.VMEM((2, IBLK, CHUNK), jnp.int32),
            pltpu.VMEM((2, IBLK, CHUNK), jnp.int32),
            pltpu.VMEM((CHUNK, D), jnp.float32),
            pltpu.VMEM((CHUNK, D), jnp.float32),
            pltpu.VMEM_SHARED((N_PAD, D), jnp.float32),
            pltpu.SemaphoreType.DMA,
            pltpu.SemaphoreType.DMA,
            pltpu.SemaphoreType.DMA,
        ],
    )
    def sc_kernel(e_hbm, x_hbm, out_hbm,
                  src_v, dst_v, bufa, bufb, agg, sema, semb, semi):
        cid = lax.axis_index("c")
        sid = lax.axis_index("s")
        tid = cid * NS + sid
        src_hbm = e_hbm.at[0]
        dst_hbm = e_hbm.at[1]

        base = tid * CHUNKS_T
        pltpu.sync_copy(src_hbm.at[pl.ds(base, IBLK)], src_v.at[0])
        pltpu.sync_copy(dst_hbm.at[pl.ds(base, IBLK)], dst_v.at[0])
        pltpu.async_copy(x_hbm.at[src_v.at[0, 0]], bufa, sema)

        @pl.loop(0, CHUNK)
        def _(i):
            for k in range(D // 16):
                bufb[i, pl.ds(k * 16, 16)] = jnp.zeros((16,), jnp.float32)

        zbase = sid * ZROWS
        nfull = ZROWS // CHUNK
        for z in range(nfull):
            pltpu.sync_copy(bufb, agg.at[pl.ds(zbase + z * CHUNK, CHUNK)])
        rem = ZROWS - nfull * CHUNK
        if rem:
            pltpu.sync_copy(bufb.at[pl.ds(0, rem)],
                            agg.at[pl.ds(zbase + nfull * CHUNK, rem)])
        plsc.subcore_barrier()

        for blk, blen in enumerate(BLOCKS):
            cur = blk % 2
            nxt = 1 - cur
            if blk + 1 < len(BLOCKS):
                hs = pltpu.async_copy(
                    src_hbm.at[pl.ds(base + (blk + 1) * IBLK, IBLK)],
                    src_v.at[nxt], semi)
                hd = pltpu.async_copy(
                    dst_hbm.at[pl.ds(base + (blk + 1) * IBLK, IBLK)],
                    dst_v.at[nxt], semi)
            sv = src_v.at[cur]
            dv = dst_v.at[cur]
            if blk > 0:
                pltpu.async_copy(x_hbm.at[sv.at[0]], bufa, sema)

            @pl.loop(0, blen // 2)
            def _(g):
                j0 = g * 2
                j1 = j0 + 1
                pltpu.async_copy(x_hbm.at[sv.at[j1]], bufb, semb)
                pltpu.make_async_copy(x_hbm.at[sv.at[j0]], bufa, sema).wait()
                pltpu.sync_copy(bufa, agg.at[dv.at[j0]], add=True)

                @pl.when(j1 + 1 < blen)
                def _():
                    pltpu.async_copy(x_hbm.at[sv.at[j1 + 1]], bufa, sema)

                pltpu.make_async_copy(x_hbm.at[sv.at[j1]], bufb, semb).wait()
                pltpu.sync_copy(bufb, agg.at[dv.at[j1]], add=True)

            if blk + 1 < len(BLOCKS):
                hs.wait()
                hd.wait()

        @pl.when(tid < NEXTRA)
        def _():
            pltpu.sync_copy(src_hbm.at[pl.ds(XBASE, NEXTRA)],
                            src_v.at[0, pl.ds(0, NEXTRA)])
            pltpu.sync_copy(dst_hbm.at[pl.ds(XBASE, NEXTRA)],
                            dst_v.at[0, pl.ds(0, NEXTRA)])
            pltpu.sync_copy(x_hbm.at[src_v.at[0, tid]], bufa)
            pltpu.sync_copy(bufa, agg.at[dst_v.at[0, tid]], add=True)

        plsc.subcore_barrier()

        obase = sid * ZROWS
        hbase = cid * N_PAD + obase
        pieces = [(p * CHUNK, CHUNK) for p in range(ZROWS // CHUNK)]
        if ZROWS % CHUNK:
            pieces.append((ZROWS - ZROWS % CHUNK, ZROWS % CHUNK))
        bufp = (bufa, bufb)
        semp = (sema, semb)
        for p, (off, ln) in enumerate(pieces):
            pb = bufp[p % 2].at[pl.ds(0, ln)]
            if p >= 2:
                poff, pln = pieces[p - 2]
                pltpu.make_async_copy(
                    bufp[p % 2].at[pl.ds(0, pln)],
                    out_hbm.at[pl.ds(hbase + poff, pln)], semi).wait()
            pltpu.sync_copy(agg.at[pl.ds(obase + off, ln)], pb)
            pltpu.async_copy(pb, out_hbm.at[pl.ds(hbase + off, ln)], semi)
        for p in range(max(len(pieces) - 2, 0), len(pieces)):
            off, ln = pieces[p]
            pltpu.make_async_copy(
                bufp[p % 2].at[pl.ds(0, ln)],
                out_hbm.at[pl.ds(hbase + off, ln)], semi).wait()

    return sc_kernel(edges3, x)


def _tc_body(x_ref, a0_ref, a1_ref, w_ref, b_ref, o_ref):
    h = x_ref[...] + a0_ref[0] + a1_ref[0]
    o_ref[...] = lax.dot_general(
        h, w_ref[...],
        dimension_numbers=(((1,), (1,)), ((), ())),
        preferred_element_type=jnp.float32,
    ) + b_ref[...]


def kernel(x, edge_index, W, b):
    edges3 = edge_index.reshape(2, NROWS, CHUNK)

    agg = _sc_aggregate(edges3, x).reshape(NC, N_PAD, D)

    BM = 2000
    nb = N_NODES // BM
    out = pl.pallas_call(
        _tc_body,
        grid=(nb,),
        in_specs=[
            pl.BlockSpec((BM, D), lambda i: (i, 0)),
            pl.BlockSpec((1, BM, D), lambda i: (0, i, 0)),
            pl.BlockSpec((1, BM, D), lambda i: (1, i, 0)),
            pl.BlockSpec((D, D), lambda i: (0, 0)),
            pl.BlockSpec((1, D), lambda i: (0, 0)),
        ],
        out_specs=pl.BlockSpec((BM, D), lambda i: (i, 0)),
        out_shape=jax.ShapeDtypeStruct((N_NODES, D), jnp.float32),
    )(x, agg, agg, W, b.reshape(1, D))
    return out
